# baseline (speedup 1.0000x reference)
import functools

import jax
import jax.numpy as jnp
from jax import lax
from jax.experimental import pallas as pl
from jax.experimental.pallas import tpu as pltpu
from jax.experimental.pallas import tpu_sc as plsc

N = 10000
E = 320000
D_IN = 128
H = 8
F = 8
HF = H * F
B = 128

NPAD = 10240
NC = 2
NS = 16
NW = NC * NS
K = 128
J = 80
EW = J * K
EPAD = NW * EW
ROWS_PER_TILE = NPAD // NS
ZR = 64

FW = 80

f32 = jnp.float32
i32 = jnp.int32


def _prep1_body(xb, w1, rw, al, ar, featl_o, er_o, res_o):
    x = xb[...]
    fb = jnp.dot(x, w1[...], preferred_element_type=f32)
    el = jnp.dot(fb, al[...], preferred_element_type=f32, precision=lax.Precision.HIGHEST)
    er = jnp.dot(fb, ar[...], preferred_element_type=f32, precision=lax.Precision.HIGHEST)
    z8 = jnp.zeros((x.shape[0], H), f32)
    featl_o[...] = jnp.concatenate([fb, el, z8], axis=1)
    er_o[...] = jnp.concatenate([er, z8], axis=1)
    res_o[...] = jnp.dot(x, rw[...], preferred_element_type=f32)


_prep1_call = pl.pallas_call(
    _prep1_body,
    grid=(NPAD // 1024,),
    in_specs=[
        pl.BlockSpec((1024, D_IN), lambda i: (i, 0)),
        pl.BlockSpec((D_IN, HF), lambda i: (0, 0)),
        pl.BlockSpec((D_IN, HF), lambda i: (0, 0)),
        pl.BlockSpec((HF, H), lambda i: (0, 0)),
        pl.BlockSpec((HF, H), lambda i: (0, 0)),
    ],
    out_specs=[
        pl.BlockSpec((1024, FW), lambda i: (i, 0)),
        pl.BlockSpec((1024, 16), lambda i: (i, 0)),
        pl.BlockSpec((1024, HF), lambda i: (i, 0)),
    ],
    out_shape=[
        jax.ShapeDtypeStruct((NPAD, FW), f32),
        jax.ShapeDtypeStruct((NPAD, 16), f32),
        jax.ShapeDtypeStruct((NPAD, HF), f32),
    ],
)


def _edge_body(featl, er16, srcp, dstp, out0, out1,
               acc, srcv, dstv, fsrc0, fsrc1, erd0, erd1, zbuf,
               sa0, sb0, sa1, sb1):
    cid = lax.axis_index("c")
    sid = lax.axis_index("s")
    wid = sid * NC + cid

    pltpu.sync_copy(srcp.at[wid], srcv)
    pltpu.sync_copy(dstp.at[wid], dstv)

    def _zrow(i, c):
        for t in range(FW // 16):
            zbuf[i, pl.ds(16 * t, 16)] = jnp.zeros((16,), f32)
        return c
    lax.fori_loop(0, ZR, _zrow, 0)

    def _zcopy(i, c):
        pltpu.sync_copy(zbuf, acc.at[pl.ds(sid * ROWS_PER_TILE + i * ZR, ZR)])
        return c
    lax.fori_loop(0, ROWS_PER_TILE // ZR, _zcopy, 0)
    plsc.subcore_barrier()

    iota = lax.iota(i32, 16)
    fbufs = (fsrc0, fsrc1)
    ebufs = (erd0, erd1)
    sas = (sa0, sa1)
    sbs = (sb0, sb1)

    def _issue(j, p):
        pltpu.async_copy(featl.at[srcv.at[j]], fbufs[p], sas[p])
        pltpu.async_copy(er16.at[dstv.at[j]], ebufs[p], sbs[p])

    def _wait(j, p):
        pltpu.make_async_copy(featl.at[srcv.at[j]], fbufs[p], sas[p]).wait()
        pltpu.make_async_copy(er16.at[dstv.at[j]], ebufs[p], sbs[p]).wait()

    def _compute(p):
        fsrc = fbufs[p]
        erd = ebufs[p]
        for g in range(K // 16):
            rows = iota + 16 * g
            for h in range(H):
                colh = jnp.full((16,), 64 + h, i32)
                elv = plsc.load_gather(fsrc, [rows, colh])
                erv = plsc.load_gather(erd, [rows, jnp.full((16,), h, i32)])
                e = elv + erv
                e = jnp.where(e < 0, e * 0.2, e)
                exv = jnp.exp(e)
                plsc.store_scatter(fsrc, [rows, colh], exv)
                for c in range(h * F, (h + 1) * F):
                    colc = jnp.full((16,), c, i32)
                    fv = plsc.load_gather(fsrc, [rows, colc])
                    plsc.store_scatter(fsrc, [rows, colc], fv * exv)

    _issue(0, 0)

    def _step(s_, carry):
        for p in range(2):
            j = 2 * s_ + p

            @pl.when(j + 1 < J)
            def _():
                _issue(j + 1, 1 - p)
            _wait(j, p)
            _compute(p)
            pltpu.sync_copy(fbufs[p], acc.at[dstv.at[j]], add=True)
        return carry
    lax.fori_loop(0, J // 2, _step, 0)

    plsc.subcore_barrier()
    rows_out = pl.ds(sid * ROWS_PER_TILE, ROWS_PER_TILE)

    @pl.when(cid == 0)
    def _():
        pltpu.sync_copy(acc.at[rows_out], out0.at[rows_out])

    @pl.when(cid == 1)
    def _():
        pltpu.sync_copy(acc.at[rows_out], out1.at[rows_out])


_edge_call = pl.kernel(
    _edge_body,
    out_type=(
        jax.ShapeDtypeStruct((NPAD, FW), f32),
        jax.ShapeDtypeStruct((NPAD, FW), f32),
    ),
    mesh=plsc.VectorSubcoreMesh(core_axis_name="c", subcore_axis_name="s",
                                num_cores=NC, num_subcores=NS),
    scratch_types=[
        pltpu.VMEM_SHARED((NPAD, FW), f32),
        pltpu.VMEM((J, K), i32),
        pltpu.VMEM((J, K), i32),
        pltpu.VMEM((K, FW), f32),
        pltpu.VMEM((K, FW), f32),
        pltpu.VMEM((K, 16), f32),
        pltpu.VMEM((K, 16), f32),
        pltpu.VMEM((ZR, FW), f32),
        pltpu.SemaphoreType.DMA,
        pltpu.SemaphoreType.DMA,
        pltpu.SemaphoreType.DMA,
        pltpu.SemaphoreType.DMA,
    ],
    compiler_params=pltpu.CompilerParams(use_tc_tiling_on_sc=False,
                                         needs_layout_passes=False),
)


def _mid_body(a0, a1, resb, b1, w2, al, ar, eb, featl_o, er_o, h_o):
    s = a0[...] + a1[...]
    den64 = jnp.dot(s[:, HF:HF + H], eb[...], preferred_element_type=f32, precision=lax.Precision.HIGHEST)
    rst = s[:, :HF] / (den64 + 1e-9)
    t = rst + resb[...] + b1[...]
    h = jnp.where(t > 0, t, jnp.exp(t) - 1.0)
    fb = jnp.dot(h, w2[...], preferred_element_type=f32)
    el = jnp.dot(fb, al[...], preferred_element_type=f32, precision=lax.Precision.HIGHEST)
    er = jnp.dot(fb, ar[...], preferred_element_type=f32, precision=lax.Precision.HIGHEST)
    z8 = jnp.zeros((h.shape[0], H), f32)
    featl_o[...] = jnp.concatenate([fb, el, z8], axis=1)
    er_o[...] = jnp.concatenate([er, z8], axis=1)
    h_o[...] = h


_mid_call = pl.pallas_call(
    _mid_body,
    grid=(NPAD // 1024,),
    in_specs=[
        pl.BlockSpec((1024, FW), lambda i: (i, 0)),
        pl.BlockSpec((1024, FW), lambda i: (i, 0)),
        pl.BlockSpec((1024, HF), lambda i: (i, 0)),
        pl.BlockSpec((1, HF), lambda i: (0, 0)),
        pl.BlockSpec((HF, HF), lambda i: (0, 0)),
        pl.BlockSpec((HF, H), lambda i: (0, 0)),
        pl.BlockSpec((HF, H), lambda i: (0, 0)),
        pl.BlockSpec((H, HF), lambda i: (0, 0)),
    ],
    out_specs=[
        pl.BlockSpec((1024, FW), lambda i: (i, 0)),
        pl.BlockSpec((1024, 16), lambda i: (i, 0)),
        pl.BlockSpec((1024, HF), lambda i: (i, 0)),
    ],
    out_shape=[
        jax.ShapeDtypeStruct((NPAD, FW), f32),
        jax.ShapeDtypeStruct((NPAD, 16), f32),
        jax.ShapeDtypeStruct((NPAD, HF), f32),
    ],
)


def _readout_body(b0, b1, hb, idsb, b2, eb, m8, wg, bg, wh, bh,
                  gam, bet, wo, bo, out_o, wsum_acc, hmaxT_acc):
    step = pl.program_id(0)

    @pl.when(step == 0)
    def _():
        wsum_acc[...] = jnp.zeros((B, H), f32)
        hmaxT_acc[...] = jnp.full((H, B), -3e38, f32)

    s = b0[...] + b1[...]
    den64 = jnp.dot(s[:, HF:HF + H], eb[...], preferred_element_type=f32, precision=lax.Precision.HIGHEST)
    rst = s[:, :HF] / (den64 + 1e-9)
    t = rst + hb[...] + b2[...]
    tm = jnp.dot(t, m8[...], preferred_element_type=f32, precision=lax.Precision.HIGHEST)
    h2 = jnp.where(tm > 0, tm, jnp.exp(tm) - 1.0)
    gl = jnp.dot(h2, wg[...], preferred_element_type=f32) + bg[...]
    gate = 1.0 / (1.0 + jnp.exp(-gl))
    hg = h2 * gate

    ids = idsb[...]
    bio = lax.broadcasted_iota(i32, (ids.shape[0], B), 1)
    oh = bio == ids
    ohf = oh.astype(f32)
    wsum_acc[...] += lax.dot_general(
        ohf, hg, (((0,), (0,)), ((), ())), preferred_element_type=f32, precision=lax.Precision.HIGHEST)

    cols = []
    for f_ in range(H):
        hf = h2[:, f_:f_ + 1]
        cols.append(jnp.max(jnp.where(oh, hf, -3e38), axis=0, keepdims=True))
    hmaxT_acc[...] = jnp.maximum(hmaxT_acc[...], jnp.concatenate(cols, axis=0))

    wsum = wsum_acc[...]
    hmT = hmaxT_acc[...]
    hmT = jnp.where(hmT <= -1e37, 0.0, hmT)
    g = jnp.concatenate([wsum, hmT.T], axis=1)
    z = jnp.dot(g, wh[...], preferred_element_type=f32) + bh[...]
    z = jnp.maximum(z, 0.0)
    mu = jnp.mean(z, axis=0, keepdims=True)
    var = jnp.mean((z - mu) * (z - mu), axis=0, keepdims=True)
    zn = (z - mu) / jnp.sqrt(var + 1e-5) * gam[...] + bet[...]
    out_o[...] = jnp.dot(zn, wo[...], preferred_element_type=f32) + bo[...]


_readout_call = pl.pallas_call(
    _readout_body,
    grid=(N // 1000,),
    in_specs=[
        pl.BlockSpec((1000, FW), lambda i: (i, 0)),
        pl.BlockSpec((1000, FW), lambda i: (i, 0)),
        pl.BlockSpec((1000, HF), lambda i: (i, 0)),
        pl.BlockSpec((1000, 1), lambda i: (i, 0)),
        pl.BlockSpec((1, HF), lambda i: (0, 0)),
        pl.BlockSpec((H, HF), lambda i: (0, 0)),
        pl.BlockSpec((HF, H), lambda i: (0, 0)),
        pl.BlockSpec((H, 1), lambda i: (0, 0)),
        pl.BlockSpec((1, 1), lambda i: (0, 0)),
        pl.BlockSpec((2 * H, B), lambda i: (0, 0)),
        pl.BlockSpec((1, B), lambda i: (0, 0)),
        pl.BlockSpec((1, B), lambda i: (0, 0)),
        pl.BlockSpec((1, B), lambda i: (0, 0)),
        pl.BlockSpec((B, 1), lambda i: (0, 0)),
        pl.BlockSpec((1, 1), lambda i: (0, 0)),
    ],
    out_specs=pl.BlockSpec((B, 1), lambda i: (0, 0)),
    out_shape=jax.ShapeDtypeStruct((B, 1), f32),
    scratch_shapes=[
        pltpu.VMEM((B, H), f32),
        pltpu.VMEM((H, B), f32),
    ],
)


def _expand_attn(a):
    return (a[:, :, None] * jnp.eye(H, dtype=f32)[:, None, :]).reshape(HF, H)


def kernel(x, edge_index, graph_ids, W1, attn_l1, attn_r1, bias1, res_W1,
           W2, attn_l2, attn_r2, bias2, w_gate, b_gate, W_h, b_h,
           gamma, beta, W_out, b_out):
    al1 = _expand_attn(attn_l1)
    ar1 = _expand_attn(attn_r1)
    al2 = _expand_attn(attn_l2)
    ar2 = _expand_attn(attn_r2)
    eb = jnp.repeat(jnp.eye(H, dtype=f32), F, axis=1)
    m8 = jnp.tile(jnp.eye(F, dtype=f32), (H, 1)) / H

    xpad = jnp.pad(x, ((0, NPAD - N), (0, 0)))
    pad_e = jnp.full((2, EPAD - E), N, i32)
    ei = jnp.concatenate([edge_index, pad_e], axis=1)
    srcp = ei[0].reshape(NW, J, K)
    dstp = ei[1].reshape(NW, J, K)
    idsc = graph_ids.reshape(N, 1)

    featl1, er16_1, res1 = _prep1_call(xpad, W1, res_W1, al1, ar1)
    a0, a1 = _edge_call(featl1, er16_1, srcp, dstp)
    featl2, er16_2, hres = _mid_call(a0, a1, res1, bias1.reshape(1, HF),
                                     W2, al2, ar2, eb)
    c0, c1 = _edge_call(featl2, er16_2, srcp, dstp)
    out = _readout_call(c0, c1, hres, idsc, bias2.reshape(1, HF), eb, m8,
                        w_gate, b_gate.reshape(1, 1), W_h,
                        b_h.reshape(1, B), gamma.reshape(1, B),
                        beta.reshape(1, B), W_out, b_out.reshape(1, 1))
    return out


# speedup vs baseline: 1.7407x; 1.7407x over previous
import functools

import jax
import jax.numpy as jnp
from jax import lax
from jax.experimental import pallas as pl
from jax.experimental.pallas import tpu as pltpu
from jax.experimental.pallas import tpu_sc as plsc

N = 10000
E = 320000
D_IN = 128
H = 8
F = 8
HF = H * F
B = 128

NPAD = 10240
NC = 2
NS = 16
NW = NC * NS
K = 128
J = 80
EW = J * K
EPAD = NW * EW
ROWS_PER_TILE = NPAD // NS
ZR = 64

FW = 80

f32 = jnp.float32
i32 = jnp.int32


def _prep1_body(xb, w1, rw, al, ar, featl_o, er_o, res_o):
    x = xb[...]
    fb = jnp.dot(x, w1[...], preferred_element_type=f32)
    el = jnp.dot(fb, al[...], preferred_element_type=f32, precision=lax.Precision.HIGHEST)
    er = jnp.dot(fb, ar[...], preferred_element_type=f32, precision=lax.Precision.HIGHEST)
    z8 = jnp.zeros((x.shape[0], H), f32)
    featl_o[...] = jnp.concatenate([fb, el, z8], axis=1)
    er_o[...] = jnp.concatenate([er, z8], axis=1)
    res_o[...] = jnp.dot(x, rw[...], preferred_element_type=f32)


_prep1_call = pl.pallas_call(
    _prep1_body,
    grid=(NPAD // 1024,),
    in_specs=[
        pl.BlockSpec((1024, D_IN), lambda i: (i, 0)),
        pl.BlockSpec((D_IN, HF), lambda i: (0, 0)),
        pl.BlockSpec((D_IN, HF), lambda i: (0, 0)),
        pl.BlockSpec((HF, H), lambda i: (0, 0)),
        pl.BlockSpec((HF, H), lambda i: (0, 0)),
    ],
    out_specs=[
        pl.BlockSpec((1024, FW), lambda i: (i, 0)),
        pl.BlockSpec((1024, 16), lambda i: (i, 0)),
        pl.BlockSpec((1024, HF), lambda i: (i, 0)),
    ],
    out_shape=[
        jax.ShapeDtypeStruct((NPAD, FW), f32),
        jax.ShapeDtypeStruct((NPAD, 16), f32),
        jax.ShapeDtypeStruct((NPAD, HF), f32),
    ],
)


def _edge_body(featl, er16, srcp, dstp, out0, out1,
               acc, srcv, dstv, fsrc0, fsrc1, erd0, erd1, zbuf,
               sa0, sb0, sa1, sb1):
    cid = lax.axis_index("c")
    sid = lax.axis_index("s")
    wid = sid * NC + cid

    pltpu.sync_copy(srcp.at[wid], srcv)
    pltpu.sync_copy(dstp.at[wid], dstv)

    def _zrow(i, c):
        for t in range(FW // 16):
            zbuf[i, pl.ds(16 * t, 16)] = jnp.zeros((16,), f32)
        return c
    lax.fori_loop(0, ZR, _zrow, 0)

    def _zcopy(i, c):
        pltpu.sync_copy(zbuf, acc.at[pl.ds(sid * ROWS_PER_TILE + i * ZR, ZR)])
        return c
    lax.fori_loop(0, ROWS_PER_TILE // ZR, _zcopy, 0)
    plsc.subcore_barrier()

    iota = lax.iota(i32, 16)
    fbufs = (fsrc0, fsrc1)
    ebufs = (erd0, erd1)
    sas = (sa0, sa1)
    sbs = (sb0, sb1)

    def _issue(j, p):
        pltpu.async_copy(featl.at[srcv.at[j]], fbufs[p], sas[p])
        pltpu.async_copy(er16.at[dstv.at[j]], ebufs[p], sbs[p])

    def _wait(j, p):
        pltpu.make_async_copy(featl.at[srcv.at[j]], fbufs[p], sas[p]).wait()
        pltpu.make_async_copy(er16.at[dstv.at[j]], ebufs[p], sbs[p]).wait()

    mask8 = iota < 8

    def _compute(p):
        fsrc = fbufs[p]
        erd = ebufs[p]

        @plsc.parallel_loop(0, K, step=1, unroll=8)
        def _edge(k):
            elv = fsrc[k, pl.ds(64, 16)]
            erv = erd[k, pl.ds(0, 16)]
            e = elv + erv
            e = jnp.where(e < 0, e * 0.2, e)
            ex = jnp.where(mask8, jnp.exp(e), 0.0)
            fsrc[k, pl.ds(64, 16)] = ex
            for t in range(4):
                fv = fsrc[k, pl.ds(16 * t, 16)]
                exb = jnp.where(mask8, ex[2 * t], ex[2 * t + 1])
                fsrc[k, pl.ds(16 * t, 16)] = fv * exb

    _issue(0, 0)

    def _step(s_, carry):
        for p in range(2):
            j = 2 * s_ + p

            @pl.when(j + 1 < J)
            def _():
                _issue(j + 1, 1 - p)
            _wait(j, p)
            _compute(p)
            pltpu.sync_copy(fbufs[p], acc.at[dstv.at[j]], add=True)
        return carry
    lax.fori_loop(0, J // 2, _step, 0)

    plsc.subcore_barrier()
    rows_out = pl.ds(sid * ROWS_PER_TILE, ROWS_PER_TILE)

    @pl.when(cid == 0)
    def _():
        pltpu.sync_copy(acc.at[rows_out], out0.at[rows_out])

    @pl.when(cid == 1)
    def _():
        pltpu.sync_copy(acc.at[rows_out], out1.at[rows_out])


_edge_call = pl.kernel(
    _edge_body,
    out_type=(
        jax.ShapeDtypeStruct((NPAD, FW), f32),
        jax.ShapeDtypeStruct((NPAD, FW), f32),
    ),
    mesh=plsc.VectorSubcoreMesh(core_axis_name="c", subcore_axis_name="s",
                                num_cores=NC, num_subcores=NS),
    scratch_types=[
        pltpu.VMEM_SHARED((NPAD, FW), f32),
        pltpu.VMEM((J, K), i32),
        pltpu.VMEM((J, K), i32),
        pltpu.VMEM((K, FW), f32),
        pltpu.VMEM((K, FW), f32),
        pltpu.VMEM((K, 16), f32),
        pltpu.VMEM((K, 16), f32),
        pltpu.VMEM((ZR, FW), f32),
        pltpu.SemaphoreType.DMA,
        pltpu.SemaphoreType.DMA,
        pltpu.SemaphoreType.DMA,
        pltpu.SemaphoreType.DMA,
    ],
    compiler_params=pltpu.CompilerParams(use_tc_tiling_on_sc=False,
                                         needs_layout_passes=False),
)


def _mid_body(a0, a1, resb, b1, w2, al, ar, eb, featl_o, er_o, h_o):
    s = a0[...] + a1[...]
    den64 = jnp.dot(s[:, HF:HF + H], eb[...], preferred_element_type=f32, precision=lax.Precision.HIGHEST)
    rst = s[:, :HF] / (den64 + 1e-9)
    t = rst + resb[...] + b1[...]
    h = jnp.where(t > 0, t, jnp.exp(t) - 1.0)
    fb = jnp.dot(h, w2[...], preferred_element_type=f32)
    el = jnp.dot(fb, al[...], preferred_element_type=f32, precision=lax.Precision.HIGHEST)
    er = jnp.dot(fb, ar[...], preferred_element_type=f32, precision=lax.Precision.HIGHEST)
    z8 = jnp.zeros((h.shape[0], H), f32)
    featl_o[...] = jnp.concatenate([fb, el, z8], axis=1)
    er_o[...] = jnp.concatenate([er, z8], axis=1)
    h_o[...] = h


_mid_call = pl.pallas_call(
    _mid_body,
    grid=(NPAD // 1024,),
    in_specs=[
        pl.BlockSpec((1024, FW), lambda i: (i, 0)),
        pl.BlockSpec((1024, FW), lambda i: (i, 0)),
        pl.BlockSpec((1024, HF), lambda i: (i, 0)),
        pl.BlockSpec((1, HF), lambda i: (0, 0)),
        pl.BlockSpec((HF, HF), lambda i: (0, 0)),
        pl.BlockSpec((HF, H), lambda i: (0, 0)),
        pl.BlockSpec((HF, H), lambda i: (0, 0)),
        pl.BlockSpec((H, HF), lambda i: (0, 0)),
    ],
    out_specs=[
        pl.BlockSpec((1024, FW), lambda i: (i, 0)),
        pl.BlockSpec((1024, 16), lambda i: (i, 0)),
        pl.BlockSpec((1024, HF), lambda i: (i, 0)),
    ],
    out_shape=[
        jax.ShapeDtypeStruct((NPAD, FW), f32),
        jax.ShapeDtypeStruct((NPAD, 16), f32),
        jax.ShapeDtypeStruct((NPAD, HF), f32),
    ],
)


def _readout_body(b0, b1, hb, idsb, b2, eb, m8, wg, bg, wh, bh,
                  gam, bet, wo, bo, out_o, wsum_acc, hmaxT_acc):
    step = pl.program_id(0)

    @pl.when(step == 0)
    def _():
        wsum_acc[...] = jnp.zeros((B, H), f32)
        hmaxT_acc[...] = jnp.full((H, B), -3e38, f32)

    s = b0[...] + b1[...]
    den64 = jnp.dot(s[:, HF:HF + H], eb[...], preferred_element_type=f32, precision=lax.Precision.HIGHEST)
    rst = s[:, :HF] / (den64 + 1e-9)
    t = rst + hb[...] + b2[...]
    tm = jnp.dot(t, m8[...], preferred_element_type=f32, precision=lax.Precision.HIGHEST)
    h2 = jnp.where(tm > 0, tm, jnp.exp(tm) - 1.0)
    gl = jnp.dot(h2, wg[...], preferred_element_type=f32) + bg[...]
    gate = 1.0 / (1.0 + jnp.exp(-gl))
    hg = h2 * gate

    ids = idsb[...]
    bio = lax.broadcasted_iota(i32, (ids.shape[0], B), 1)
    oh = bio == ids
    ohf = oh.astype(f32)
    wsum_acc[...] += lax.dot_general(
        ohf, hg, (((0,), (0,)), ((), ())), preferred_element_type=f32, precision=lax.Precision.HIGHEST)

    cols = []
    for f_ in range(H):
        hf = h2[:, f_:f_ + 1]
        cols.append(jnp.max(jnp.where(oh, hf, -3e38), axis=0, keepdims=True))
    hmaxT_acc[...] = jnp.maximum(hmaxT_acc[...], jnp.concatenate(cols, axis=0))

    wsum = wsum_acc[...]
    hmT = hmaxT_acc[...]
    hmT = jnp.where(hmT <= -1e37, 0.0, hmT)
    g = jnp.concatenate([wsum, hmT.T], axis=1)
    z = jnp.dot(g, wh[...], preferred_element_type=f32) + bh[...]
    z = jnp.maximum(z, 0.0)
    mu = jnp.mean(z, axis=0, keepdims=True)
    var = jnp.mean((z - mu) * (z - mu), axis=0, keepdims=True)
    zn = (z - mu) / jnp.sqrt(var + 1e-5) * gam[...] + bet[...]
    out_o[...] = jnp.dot(zn, wo[...], preferred_element_type=f32) + bo[...]


_readout_call = pl.pallas_call(
    _readout_body,
    grid=(N // 1000,),
    in_specs=[
        pl.BlockSpec((1000, FW), lambda i: (i, 0)),
        pl.BlockSpec((1000, FW), lambda i: (i, 0)),
        pl.BlockSpec((1000, HF), lambda i: (i, 0)),
        pl.BlockSpec((1000, 1), lambda i: (i, 0)),
        pl.BlockSpec((1, HF), lambda i: (0, 0)),
        pl.BlockSpec((H, HF), lambda i: (0, 0)),
        pl.BlockSpec((HF, H), lambda i: (0, 0)),
        pl.BlockSpec((H, 1), lambda i: (0, 0)),
        pl.BlockSpec((1, 1), lambda i: (0, 0)),
        pl.BlockSpec((2 * H, B), lambda i: (0, 0)),
        pl.BlockSpec((1, B), lambda i: (0, 0)),
        pl.BlockSpec((1, B), lambda i: (0, 0)),
        pl.BlockSpec((1, B), lambda i: (0, 0)),
        pl.BlockSpec((B, 1), lambda i: (0, 0)),
        pl.BlockSpec((1, 1), lambda i: (0, 0)),
    ],
    out_specs=pl.BlockSpec((B, 1), lambda i: (0, 0)),
    out_shape=jax.ShapeDtypeStruct((B, 1), f32),
    scratch_shapes=[
        pltpu.VMEM((B, H), f32),
        pltpu.VMEM((H, B), f32),
    ],
)


def _expand_attn(a):
    return (a[:, :, None] * jnp.eye(H, dtype=f32)[:, None, :]).reshape(HF, H)


def kernel(x, edge_index, graph_ids, W1, attn_l1, attn_r1, bias1, res_W1,
           W2, attn_l2, attn_r2, bias2, w_gate, b_gate, W_h, b_h,
           gamma, beta, W_out, b_out):
    al1 = _expand_attn(attn_l1)
    ar1 = _expand_attn(attn_r1)
    al2 = _expand_attn(attn_l2)
    ar2 = _expand_attn(attn_r2)
    eb = jnp.repeat(jnp.eye(H, dtype=f32), F, axis=1)
    m8 = jnp.tile(jnp.eye(F, dtype=f32), (H, 1)) / H

    xpad = jnp.pad(x, ((0, NPAD - N), (0, 0)))
    pad_e = jnp.full((2, EPAD - E), N, i32)
    ei = jnp.concatenate([edge_index, pad_e], axis=1)
    srcp = ei[0].reshape(NW, J, K)
    dstp = ei[1].reshape(NW, J, K)
    idsc = graph_ids.reshape(N, 1)

    featl1, er16_1, res1 = _prep1_call(xpad, W1, res_W1, al1, ar1)
    a0, a1 = _edge_call(featl1, er16_1, srcp, dstp)
    featl2, er16_2, hres = _mid_call(a0, a1, res1, bias1.reshape(1, HF),
                                     W2, al2, ar2, eb)
    c0, c1 = _edge_call(featl2, er16_2, srcp, dstp)
    out = _readout_call(c0, c1, hres, idsc, bias2.reshape(1, HF), eb, m8,
                        w_gate, b_gate.reshape(1, 1), W_h,
                        b_h.reshape(1, B), gamma.reshape(1, B),
                        beta.reshape(1, B), W_out, b_out.reshape(1, 1))
    return out


# speedup vs baseline: 1.7485x; 1.0045x over previous
import functools

import jax
import jax.numpy as jnp
from jax import lax
from jax.experimental import pallas as pl
from jax.experimental.pallas import tpu as pltpu
from jax.experimental.pallas import tpu_sc as plsc

N = 10000
E = 320000
D_IN = 128
H = 8
F = 8
HF = H * F
B = 128

NPAD = 10240
NC = 2
NS = 16
NW = NC * NS
K = 128
J = 80
EW = J * K
EPAD = NW * EW
ROWS_PER_TILE = NPAD // NS
ZR = 64

FW = 80

f32 = jnp.float32
i32 = jnp.int32


def _prep1_body(xb, w1, rw, al, ar, featl_o, er_o, res_o):
    x = xb[...]
    fb = jnp.dot(x, w1[...], preferred_element_type=f32)
    el = jnp.dot(fb, al[...], preferred_element_type=f32, precision=lax.Precision.HIGHEST)
    er = jnp.dot(fb, ar[...], preferred_element_type=f32, precision=lax.Precision.HIGHEST)
    z8 = jnp.zeros((x.shape[0], H), f32)
    featl_o[...] = jnp.concatenate([fb, el, z8], axis=1)
    er_o[...] = jnp.concatenate([er, z8], axis=1)
    res_o[...] = jnp.dot(x, rw[...], preferred_element_type=f32)


_prep1_call = pl.pallas_call(
    _prep1_body,
    grid=(NPAD // 1024,),
    in_specs=[
        pl.BlockSpec((1024, D_IN), lambda i: (i, 0)),
        pl.BlockSpec((D_IN, HF), lambda i: (0, 0)),
        pl.BlockSpec((D_IN, HF), lambda i: (0, 0)),
        pl.BlockSpec((HF, H), lambda i: (0, 0)),
        pl.BlockSpec((HF, H), lambda i: (0, 0)),
    ],
    out_specs=[
        pl.BlockSpec((1024, FW), lambda i: (i, 0)),
        pl.BlockSpec((1024, 16), lambda i: (i, 0)),
        pl.BlockSpec((1024, HF), lambda i: (i, 0)),
    ],
    out_shape=[
        jax.ShapeDtypeStruct((NPAD, FW), f32),
        jax.ShapeDtypeStruct((NPAD, 16), f32),
        jax.ShapeDtypeStruct((NPAD, HF), f32),
    ],
)


def _edge_body(featl, er16, srcp, dstp, out0, out1,
               acc, srcv, dstv, fsrc0, fsrc1, erd0, erd1, zbuf,
               sa0, sb0, sa1, sb1, sw0, sw1):
    cid = lax.axis_index("c")
    sid = lax.axis_index("s")
    wid = sid * NC + cid

    pltpu.sync_copy(srcp.at[wid], srcv)
    pltpu.sync_copy(dstp.at[wid], dstv)

    def _zrow(i, c):
        for t in range(FW // 16):
            zbuf[i, pl.ds(16 * t, 16)] = jnp.zeros((16,), f32)
        return c
    lax.fori_loop(0, ZR, _zrow, 0)

    def _zcopy(i, c):
        pltpu.sync_copy(zbuf, acc.at[pl.ds(sid * ROWS_PER_TILE + i * ZR, ZR)])
        return c
    lax.fori_loop(0, ROWS_PER_TILE // ZR, _zcopy, 0)
    plsc.subcore_barrier()

    fbufs = (fsrc0, fsrc1)
    ebufs = (erd0, erd1)
    sas = (sa0, sa1)
    sbs = (sb0, sb1)
    sws = (sw0, sw1)
    mask8 = lax.iota(i32, 16) < 8

    def _issue(j, p):
        pltpu.async_copy(featl.at[srcv.at[j]], fbufs[p], sas[p])
        pltpu.async_copy(er16.at[dstv.at[j]], ebufs[p], sbs[p])

    def _wait(j, p):
        pltpu.make_async_copy(featl.at[srcv.at[j]], fbufs[p], sas[p]).wait()
        pltpu.make_async_copy(er16.at[dstv.at[j]], ebufs[p], sbs[p]).wait()

    def _wait_scatter(j, p):
        pltpu.make_async_copy(fbufs[p], acc.at[dstv.at[j]], sws[p]).wait()

    def _compute(p):
        fsrc = fbufs[p]
        erd = ebufs[p]

        @plsc.parallel_loop(0, K, step=1, unroll=8)
        def _edge(k):
            elv = fsrc[k, pl.ds(64, 16)]
            erv = erd[k, pl.ds(0, 16)]
            e = elv + erv
            e = jnp.where(e < 0, e * 0.2, e)
            ex = jnp.exp(e)
            fsrc[k, pl.ds(64, 16)] = ex
            for t in range(4):
                fv = fsrc[k, pl.ds(16 * t, 16)]
                exb = jnp.where(mask8, ex[2 * t], ex[2 * t + 1])
                fsrc[k, pl.ds(16 * t, 16)] = fv * exb

    _issue(0, 0)

    def _step(s_, carry):
        for p in range(2):
            j = 2 * s_ + p

            @pl.when(j + 1 < J)
            def _():
                _issue(j + 1, 1 - p)
            _wait(j, p)
            _compute(p)
            pltpu.sync_copy(fbufs[p], acc.at[dstv.at[j]], add=True)
        return carry
    lax.fori_loop(0, J // 2, _step, 0)

    plsc.subcore_barrier()
    rows_out = pl.ds(sid * ROWS_PER_TILE, ROWS_PER_TILE)

    @pl.when(cid == 0)
    def _():
        pltpu.sync_copy(acc.at[rows_out], out0.at[rows_out])

    @pl.when(cid == 1)
    def _():
        pltpu.sync_copy(acc.at[rows_out], out1.at[rows_out])


_edge_call = pl.kernel(
    _edge_body,
    out_type=(
        jax.ShapeDtypeStruct((NPAD, FW), f32),
        jax.ShapeDtypeStruct((NPAD, FW), f32),
    ),
    mesh=plsc.VectorSubcoreMesh(core_axis_name="c", subcore_axis_name="s",
                                num_cores=NC, num_subcores=NS),
    scratch_types=[
        pltpu.VMEM_SHARED((NPAD, FW), f32),
        pltpu.VMEM((J, K), i32),
        pltpu.VMEM((J, K), i32),
        pltpu.VMEM((K, FW), f32),
        pltpu.VMEM((K, FW), f32),
        pltpu.VMEM((K, 16), f32),
        pltpu.VMEM((K, 16), f32),
        pltpu.VMEM((ZR, FW), f32),
        pltpu.SemaphoreType.DMA,
        pltpu.SemaphoreType.DMA,
        pltpu.SemaphoreType.DMA,
        pltpu.SemaphoreType.DMA,
        pltpu.SemaphoreType.DMA,
        pltpu.SemaphoreType.DMA,
    ],
    compiler_params=pltpu.CompilerParams(use_tc_tiling_on_sc=False,
                                         needs_layout_passes=False),
)


def _mid_body(a0, a1, resb, b1, w2, al, ar, eb, featl_o, er_o, h_o):
    s = a0[...] + a1[...]
    den64 = jnp.dot(s[:, HF:HF + H], eb[...], preferred_element_type=f32, precision=lax.Precision.HIGHEST)
    rst = s[:, :HF] / (den64 + 1e-9)
    t = rst + resb[...] + b1[...]
    h = jnp.where(t > 0, t, jnp.exp(t) - 1.0)
    fb = jnp.dot(h, w2[...], preferred_element_type=f32)
    el = jnp.dot(fb, al[...], preferred_element_type=f32, precision=lax.Precision.HIGHEST)
    er = jnp.dot(fb, ar[...], preferred_element_type=f32, precision=lax.Precision.HIGHEST)
    z8 = jnp.zeros((h.shape[0], H), f32)
    featl_o[...] = jnp.concatenate([fb, el, z8], axis=1)
    er_o[...] = jnp.concatenate([er, z8], axis=1)
    h_o[...] = h


_mid_call = pl.pallas_call(
    _mid_body,
    grid=(NPAD // 1024,),
    in_specs=[
        pl.BlockSpec((1024, FW), lambda i: (i, 0)),
        pl.BlockSpec((1024, FW), lambda i: (i, 0)),
        pl.BlockSpec((1024, HF), lambda i: (i, 0)),
        pl.BlockSpec((1, HF), lambda i: (0, 0)),
        pl.BlockSpec((HF, HF), lambda i: (0, 0)),
        pl.BlockSpec((HF, H), lambda i: (0, 0)),
        pl.BlockSpec((HF, H), lambda i: (0, 0)),
        pl.BlockSpec((H, HF), lambda i: (0, 0)),
    ],
    out_specs=[
        pl.BlockSpec((1024, FW), lambda i: (i, 0)),
        pl.BlockSpec((1024, 16), lambda i: (i, 0)),
        pl.BlockSpec((1024, HF), lambda i: (i, 0)),
    ],
    out_shape=[
        jax.ShapeDtypeStruct((NPAD, FW), f32),
        jax.ShapeDtypeStruct((NPAD, 16), f32),
        jax.ShapeDtypeStruct((NPAD, HF), f32),
    ],
)


def _readout_body(b0, b1, hb, idsb, b2, eb, m8, wg, bg, wh, bh,
                  gam, bet, wo, bo, out_o, wsum_acc, hmaxT_acc):
    step = pl.program_id(0)

    @pl.when(step == 0)
    def _():
        wsum_acc[...] = jnp.zeros((B, H), f32)
        hmaxT_acc[...] = jnp.full((H, B), -3e38, f32)

    s = b0[...] + b1[...]
    den64 = jnp.dot(s[:, HF:HF + H], eb[...], preferred_element_type=f32, precision=lax.Precision.HIGHEST)
    rst = s[:, :HF] / (den64 + 1e-9)
    t = rst + hb[...] + b2[...]
    tm = jnp.dot(t, m8[...], preferred_element_type=f32, precision=lax.Precision.HIGHEST)
    h2 = jnp.where(tm > 0, tm, jnp.exp(tm) - 1.0)
    gl = jnp.dot(h2, wg[...], preferred_element_type=f32) + bg[...]
    gate = 1.0 / (1.0 + jnp.exp(-gl))
    hg = h2 * gate

    ids = idsb[...]
    bio = lax.broadcasted_iota(i32, (ids.shape[0], B), 1)
    oh = bio == ids
    ohf = oh.astype(f32)
    wsum_acc[...] += lax.dot_general(
        ohf, hg, (((0,), (0,)), ((), ())), preferred_element_type=f32, precision=lax.Precision.HIGHEST)

    cols = []
    for f_ in range(H):
        hf = h2[:, f_:f_ + 1]
        cols.append(jnp.max(jnp.where(oh, hf, -3e38), axis=0, keepdims=True))
    hmaxT_acc[...] = jnp.maximum(hmaxT_acc[...], jnp.concatenate(cols, axis=0))

    wsum = wsum_acc[...]
    hmT = hmaxT_acc[...]
    hmT = jnp.where(hmT <= -1e37, 0.0, hmT)
    g = jnp.concatenate([wsum, hmT.T], axis=1)
    z = jnp.dot(g, wh[...], preferred_element_type=f32) + bh[...]
    z = jnp.maximum(z, 0.0)
    mu = jnp.mean(z, axis=0, keepdims=True)
    var = jnp.mean((z - mu) * (z - mu), axis=0, keepdims=True)
    zn = (z - mu) / jnp.sqrt(var + 1e-5) * gam[...] + bet[...]
    out_o[...] = jnp.dot(zn, wo[...], preferred_element_type=f32) + bo[...]


_readout_call = pl.pallas_call(
    _readout_body,
    grid=(N // 1000,),
    in_specs=[
        pl.BlockSpec((1000, FW), lambda i: (i, 0)),
        pl.BlockSpec((1000, FW), lambda i: (i, 0)),
        pl.BlockSpec((1000, HF), lambda i: (i, 0)),
        pl.BlockSpec((1000, 1), lambda i: (i, 0)),
        pl.BlockSpec((1, HF), lambda i: (0, 0)),
        pl.BlockSpec((H, HF), lambda i: (0, 0)),
        pl.BlockSpec((HF, H), lambda i: (0, 0)),
        pl.BlockSpec((H, 1), lambda i: (0, 0)),
        pl.BlockSpec((1, 1), lambda i: (0, 0)),
        pl.BlockSpec((2 * H, B), lambda i: (0, 0)),
        pl.BlockSpec((1, B), lambda i: (0, 0)),
        pl.BlockSpec((1, B), lambda i: (0, 0)),
        pl.BlockSpec((1, B), lambda i: (0, 0)),
        pl.BlockSpec((B, 1), lambda i: (0, 0)),
        pl.BlockSpec((1, 1), lambda i: (0, 0)),
    ],
    out_specs=pl.BlockSpec((B, 1), lambda i: (0, 0)),
    out_shape=jax.ShapeDtypeStruct((B, 1), f32),
    scratch_shapes=[
        pltpu.VMEM((B, H), f32),
        pltpu.VMEM((H, B), f32),
    ],
)


def _expand_attn(a):
    return (a[:, :, None] * jnp.eye(H, dtype=f32)[:, None, :]).reshape(HF, H)


def kernel(x, edge_index, graph_ids, W1, attn_l1, attn_r1, bias1, res_W1,
           W2, attn_l2, attn_r2, bias2, w_gate, b_gate, W_h, b_h,
           gamma, beta, W_out, b_out):
    al1 = _expand_attn(attn_l1)
    ar1 = _expand_attn(attn_r1)
    al2 = _expand_attn(attn_l2)
    ar2 = _expand_attn(attn_r2)
    eb = jnp.repeat(jnp.eye(H, dtype=f32), F, axis=1)
    m8 = jnp.tile(jnp.eye(F, dtype=f32), (H, 1)) / H

    xpad = jnp.pad(x, ((0, NPAD - N), (0, 0)))
    pad_e = jnp.full((2, EPAD - E), N, i32)
    ei = jnp.concatenate([edge_index, pad_e], axis=1)
    srcp = ei[0].reshape(NW, J, K)
    dstp = ei[1].reshape(NW, J, K)
    idsc = graph_ids.reshape(N, 1)

    featl1, er16_1, res1 = _prep1_call(xpad, W1, res_W1, al1, ar1)
    a0, a1 = _edge_call(featl1, er16_1, srcp, dstp)
    featl2, er16_2, hres = _mid_call(a0, a1, res1, bias1.reshape(1, HF),
                                     W2, al2, ar2, eb)
    c0, c1 = _edge_call(featl2, er16_2, srcp, dstp)
    out = _readout_call(c0, c1, hres, idsc, bias2.reshape(1, HF), eb, m8,
                        w_gate, b_gate.reshape(1, 1), W_h,
                        b_h.reshape(1, B), gamma.reshape(1, B),
                        beta.reshape(1, B), W_out, b_out.reshape(1, 1))
    return out
